# Initial kernel scaffold; baseline (speedup 1.0000x reference)
import jax
import jax.numpy as jnp
from jax.experimental import pallas as pl


def kernel(x, edge_index, edge_attr, batch, W1, b1, W2, b2, Wro, bro, Wfc1, bfc1, gamma, beta, Wfc2, bfc2, Wd1, bd1, Wd2, bd2):
    raise NotImplementedError("write your pallas kernel here")


# speedup vs baseline: 16.8541x; 16.8541x over previous
import functools

import jax
import jax.numpy as jnp
from jax import lax
from jax.experimental import pallas as pl
from jax.experimental.pallas import tpu as pltpu
from jax.experimental.pallas import tpu_sc as plsc

N = 51200
E = 819200
D = 200
H = 64
B = 256

NC, NS = 2, 16
LPB = 128
CHB = 8
K = LPB * CHB
NPT = N // NS
NQ = 4
QW = H // NQ

_mesh = plsc.VectorSubcoreMesh(core_axis_name="c", subcore_axis_name="s")


@functools.partial(
    pl.kernel,
    out_type=jax.ShapeDtypeStruct((NC, N), jnp.float32),
    mesh=_mesh,
    scratch_types=[
        pltpu.VMEM((CHB, LPB), jnp.int32),
        pltpu.VMEM((CHB, LPB), jnp.float32),
        pltpu.VMEM((NPT,), jnp.float32),
        pltpu.VMEM_SHARED((N,), jnp.float32),
    ],
    compiler_params=pltpu.CompilerParams(use_tc_tiling_on_sc=False),
)
def _deg_sc(colb_hbm, wb_hbm, out_hbm, col_v, w_v, stage_v, acc_sh):
    c = lax.axis_index("c")
    s = lax.axis_index("s")
    tile = c * NS + s

    z = jnp.zeros((16,), jnp.float32)

    def z16(i, carry):
        stage_v[pl.ds(i * 16, 16)] = z
        return carry
    lax.fori_loop(0, NPT // 16, z16, 0)
    pltpu.sync_copy(stage_v, acc_sh.at[pl.ds(s * NPT, NPT)])
    plsc.subcore_barrier()

    blocks_per_tile = (E // LPB) // (NC * NS)
    nchunks = blocks_per_tile // CHB
    base_blk = tile * blocks_per_tile

    def chunk(j, carry):
        blk = base_blk + j * CHB
        pltpu.sync_copy(colb_hbm.at[pl.ds(blk, CHB)], col_v)
        pltpu.sync_copy(wb_hbm.at[pl.ds(blk, CHB)], w_v)
        for t in range(CHB):
            pltpu.sync_copy(w_v.at[t], acc_sh.at[col_v.at[t]], add=True)
        return carry
    lax.fori_loop(0, nchunks, chunk, 0)
    plsc.subcore_barrier()

    pltpu.sync_copy(acc_sh.at[pl.ds(s * NPT, NPT)], stage_v)
    pltpu.sync_copy(stage_v, out_hbm.at[c, pl.ds(s * NPT, NPT)])


@functools.partial(
    pl.kernel,
    out_type=jax.ShapeDtypeStruct((NQ, N, QW), jnp.float32),
    mesh=_mesh,
    scratch_types=[
        pltpu.VMEM((CHB, LPB), jnp.int32),
        pltpu.VMEM((CHB, LPB), jnp.int32),
        pltpu.VMEM((K,), jnp.float32),
        pltpu.VMEM((K, QW), jnp.float32),
        pltpu.VMEM_SHARED((N, QW), jnp.float32),
        pltpu.SemaphoreType.DMA,
    ],
    compiler_params=pltpu.CompilerParams(use_tc_tiling_on_sc=False),
)
def _agg_sc(rowb_hbm, colb_hbm, w_hbm, tab_hbm, out_hbm,
            row_v, col_v, w_v, rows_v, acc_sh, gsem):
    c = lax.axis_index("c")
    s = lax.axis_index("s")

    z = jnp.zeros((16,), jnp.float32)
    b0 = s * NPT
    blocks_per_tile = (E // LPB) // NS
    nchunks = blocks_per_tile // CHB
    base_blk = s * blocks_per_tile

    for q in range(2):
        v = c * 2 + q

        def zrow(i, carry):
            rows_v[i, pl.ds(0, 16)] = z
            return carry
        lax.fori_loop(0, K, zrow, 0)
        for p in range(3):
            pltpu.sync_copy(rows_v, acc_sh.at[pl.ds(b0 + p * K, K)])
        pltpu.sync_copy(rows_v.at[pl.ds(0, 128)],
                        acc_sh.at[pl.ds(b0 + 3 * K, 128)])
        plsc.subcore_barrier()

        def chunk(j, carry):
            blk = base_blk + j * CHB
            pltpu.sync_copy(rowb_hbm.at[v, pl.ds(blk, CHB)], row_v)
            pltpu.sync_copy(colb_hbm.at[pl.ds(blk, CHB)], col_v)
            pltpu.sync_copy(w_hbm.at[pl.ds(blk * LPB, K)], w_v)
            cps = []
            for t in range(CHB):
                cps.append(pltpu.async_copy(
                    tab_hbm.at[row_v.at[t]],
                    rows_v.at[pl.ds(t * LPB, LPB)], gsem))
            for cp in cps:
                cp.wait()

            def mul(g, carry2):
                base = g * 16
                w16 = w_v[pl.ds(base, 16)]
                for i in range(16):
                    e = base + i
                    r = rows_v[e, pl.ds(0, 16)] * w16[i]
                    rows_v[e, pl.ds(0, 16)] = r
                return carry2
            lax.fori_loop(0, K // 16, mul, 0)

            for t in range(CHB):
                pltpu.sync_copy(rows_v.at[pl.ds(t * LPB, LPB)],
                                acc_sh.at[col_v.at[t]], add=True)
            return carry
        lax.fori_loop(0, nchunks, chunk, 0)
        plsc.subcore_barrier()

        for p in range(3):
            pltpu.sync_copy(acc_sh.at[pl.ds(b0 + p * K, K)], rows_v)
            pltpu.sync_copy(rows_v, out_hbm.at[v, pl.ds(b0 + p * K, K)])
        pltpu.sync_copy(acc_sh.at[pl.ds(b0 + 3 * K, 128)],
                        rows_v.at[pl.ds(0, 128)])
        pltpu.sync_copy(rows_v.at[pl.ds(0, 128)],
                        out_hbm.at[v, pl.ds(b0 + 3 * K, 128)])


_BN = 3200


def _mish(v):
    return v * jnp.tanh(jnp.logaddexp(v, 0.0))


def _mm_body(x_ref, w_ref, o_ref):
    o_ref[...] = lax.dot_general(
        x_ref[...], w_ref[...], (((1,), (1,)), ((), ())),
        preferred_element_type=jnp.float32)


def _mm(x, w):
    n, d = x.shape
    fo = w.shape[0]
    return pl.pallas_call(
        _mm_body,
        grid=(n // _BN,),
        in_specs=[pl.BlockSpec((_BN, d), lambda i: (i, 0)),
                  pl.BlockSpec((fo, d), lambda i: (0, 0))],
        out_specs=pl.BlockSpec((_BN, fo), lambda i: (i, 0)),
        out_shape=jax.ShapeDtypeStruct((n, fo), jnp.float32),
    )(x, w)


def _scale_body(p_ref, h_ref, o_ref):
    dinv = lax.rsqrt(p_ref[0] + p_ref[1] + 1.0)
    hs = h_ref[...] * dinv[:, None]
    for v in range(NQ):
        o_ref[v] = hs[:, v * QW:(v + 1) * QW]


def _scale(degp, h):
    return pl.pallas_call(
        _scale_body,
        grid=(N // _BN,),
        in_specs=[pl.BlockSpec((NC, _BN), lambda i: (0, i)),
                  pl.BlockSpec((_BN, H), lambda i: (i, 0))],
        out_specs=pl.BlockSpec((NQ, _BN, QW), lambda i: (0, i, 0)),
        out_shape=jax.ShapeDtypeStruct((NQ, N, QW), jnp.float32),
    )(degp, h)


def _layer_mid_body(p_ref, g_ref, h_ref, b_ref, w2_ref, oh_ref, os_ref):
    dinv = lax.rsqrt(p_ref[0] + p_ref[1] + 1.0)
    a = jnp.concatenate([g_ref[v] for v in range(NQ)], axis=1)
    n1 = _mish(a * dinv[:, None] + h_ref[...] * (dinv * dinv)[:, None]
               + b_ref[...])
    h2 = lax.dot_general(n1, w2_ref[...], (((1,), (1,)), ((), ())),
                         preferred_element_type=jnp.float32)
    oh_ref[...] = h2
    hs2 = h2 * dinv[:, None]
    for v in range(NQ):
        os_ref[v] = hs2[:, v * QW:(v + 1) * QW]


def _layer_mid(degp, agg, h, b, w2):
    return pl.pallas_call(
        _layer_mid_body,
        grid=(N // _BN,),
        in_specs=[pl.BlockSpec((NC, _BN), lambda i: (0, i)),
                  pl.BlockSpec((NQ, _BN, QW), lambda i: (0, i, 0)),
                  pl.BlockSpec((_BN, H), lambda i: (i, 0)),
                  pl.BlockSpec((1, H), lambda i: (0, 0)),
                  pl.BlockSpec((H, H), lambda i: (0, 0))],
        out_specs=[pl.BlockSpec((_BN, H), lambda i: (i, 0)),
                   pl.BlockSpec((NQ, _BN, QW), lambda i: (0, i, 0))],
        out_shape=[jax.ShapeDtypeStruct((N, H), jnp.float32),
                   jax.ShapeDtypeStruct((NQ, N, QW), jnp.float32)],
    )(degp, agg, h, b, w2)


def _layer_out_body(p_ref, g_ref, h_ref, b_ref, wro_ref, bro_ref, o_ref):
    dinv = lax.rsqrt(p_ref[0] + p_ref[1] + 1.0)
    a = jnp.concatenate([g_ref[v] for v in range(NQ)], axis=1)
    n2 = _mish(a * dinv[:, None] + h_ref[...] * (dinv * dinv)[:, None]
               + b_ref[...])
    o_ref[...] = _mish(
        lax.dot_general(n2, wro_ref[...], (((1,), (1,)), ((), ())),
                        preferred_element_type=jnp.float32) + bro_ref[...])


def _layer_out(degp, agg, h, b, wro, bro):
    return pl.pallas_call(
        _layer_out_body,
        grid=(N // _BN,),
        in_specs=[pl.BlockSpec((NC, _BN), lambda i: (0, i)),
                  pl.BlockSpec((NQ, _BN, QW), lambda i: (0, i, 0)),
                  pl.BlockSpec((_BN, H), lambda i: (i, 0)),
                  pl.BlockSpec((1, H), lambda i: (0, 0)),
                  pl.BlockSpec((8, H), lambda i: (0, 0)),
                  pl.BlockSpec((1, 8), lambda i: (0, 0))],
        out_specs=pl.BlockSpec((_BN, 8), lambda i: (i, 0)),
        out_shape=jax.ShapeDtypeStruct((N, 8), jnp.float32),
    )(degp, agg, h, b, wro, bro)


def _head_body(f_ref, w1_ref, b1_ref, g_ref, be_ref, w2_ref, b2_ref,
               wd1_ref, bd1_ref, wd2_ref, bd2_ref, ol_ref, oc_ref):
    z = lax.dot_general(f_ref[...], w1_ref[...], (((1,), (1,)), ((), ())),
                        preferred_element_type=jnp.float32) + b1_ref[...]
    mu = jnp.mean(z, axis=0, keepdims=True)
    var = jnp.mean((z - mu) ** 2, axis=0, keepdims=True)
    zn = (z - mu) / jnp.sqrt(var + 1e-5)
    mid = _mish(zn * g_ref[...] + be_ref[...])
    ol_ref[...] = lax.dot_general(
        mid, w2_ref[...], (((1,), (1,)), ((), ())),
        preferred_element_type=jnp.float32) + b2_ref[...]
    dd = jnp.maximum(
        lax.dot_general(mid, wd1_ref[...], (((1,), (1,)), ((), ())),
                        preferred_element_type=jnp.float32) + bd1_ref[...], 0.0)
    oc_ref[...] = lax.dot_general(
        dd, wd2_ref[...], (((1,), (1,)), ((), ())),
        preferred_element_type=jnp.float32) + bd2_ref[...]


def _head(feats, wfc1, bfc1, gamma, beta, wfc2, bfc2, wd1, bd1, wd2, bd2):
    full = lambda s: pl.BlockSpec(s, lambda: tuple(0 for _ in s))
    return pl.pallas_call(
        _head_body,
        in_specs=[full((B, 8 * D)), full((D, 8 * D)), full((1, D)),
                  full((1, D)), full((1, D)), full((2, D)), full((1, 2)),
                  full((H, D)), full((1, H)), full((6, H)), full((1, 6))],
        out_specs=[full((B, 2)), full((B, 6))],
        out_shape=[jax.ShapeDtypeStruct((B, 2), jnp.float32),
                   jax.ShapeDtypeStruct((B, 6), jnp.float32)],
    )(feats, wfc1, bfc1, gamma, beta, wfc2, bfc2, wd1, bd1, wd2, bd2)


@jax.jit
def kernel(x, edge_index, edge_attr, batch, W1, b1, W2, b2, Wro, bro,
           Wfc1, bfc1, gamma, beta, Wfc2, bfc2, Wd1, bd1, Wd2, bd2):
    row = edge_index[0]
    col = edge_index[1]
    rowb = jnp.stack([row + v * N for v in range(NQ)]).reshape(
        NQ, E // LPB, LPB)
    colb = col.reshape(E // LPB, LPB)
    wb = edge_attr.reshape(E // LPB, LPB)

    degp = _deg_sc(colb, wb)
    h1 = _mm(x, W1)
    hs1 = _scale(degp, h1)
    agg1 = _agg_sc(rowb, colb, edge_attr, hs1.reshape(NQ * N, QW))
    h2, hs2 = _layer_mid(degp, agg1, h1, b1.reshape(1, H), W2)
    agg2 = _agg_sc(rowb, colb, edge_attr, hs2.reshape(NQ * N, QW))
    out8 = _layer_out(degp, agg2, h2, b2.reshape(1, H), Wro, bro.reshape(1, 8))
    feats = out8.reshape(B, 8 * D)
    logits, cls = _head(feats, Wfc1, bfc1.reshape(1, D), gamma.reshape(1, D),
                        beta.reshape(1, D), Wfc2, bfc2.reshape(1, 2),
                        Wd1, bd1.reshape(1, H), Wd2, bd2.reshape(1, 6))
    return (logits, cls)


# speedup vs baseline: 20.9228x; 1.2414x over previous
import functools

import jax
import jax.numpy as jnp
from jax import lax
from jax.experimental import pallas as pl
from jax.experimental.pallas import tpu as pltpu
from jax.experimental.pallas import tpu_sc as plsc

N = 51200
E = 819200
D = 200
H = 64
B = 256

NC, NS = 2, 16
LPB = 128
CHB = 8
K = LPB * CHB
NPT = N // NS
NQ = 4
QW = H // NQ
NCHUNK = (E // LPB) // NS // CHB

_mesh = plsc.VectorSubcoreMesh(core_axis_name="c", subcore_axis_name="s")


@functools.partial(
    pl.kernel,
    out_type=jax.ShapeDtypeStruct((NC, N), jnp.float32),
    mesh=_mesh,
    scratch_types=[
        pltpu.VMEM((CHB, LPB), jnp.int32),
        pltpu.VMEM((CHB, LPB), jnp.float32),
        pltpu.VMEM((NPT,), jnp.float32),
        pltpu.VMEM_SHARED((N,), jnp.float32),
    ],
    compiler_params=pltpu.CompilerParams(use_tc_tiling_on_sc=False),
)
def _deg_sc(colb_hbm, wb_hbm, out_hbm, col_v, w_v, stage_v, acc_sh):
    c = lax.axis_index("c")
    s = lax.axis_index("s")
    tile = c * NS + s

    z = jnp.zeros((16,), jnp.float32)

    def z16(i, carry):
        stage_v[pl.ds(i * 16, 16)] = z
        return carry
    lax.fori_loop(0, NPT // 16, z16, 0)
    pltpu.sync_copy(stage_v, acc_sh.at[pl.ds(s * NPT, NPT)])
    plsc.subcore_barrier()

    blocks_per_tile = (E // LPB) // (NC * NS)
    nchunks = blocks_per_tile // CHB
    base_blk = tile * blocks_per_tile

    def chunk(j, carry):
        blk = base_blk + j * CHB
        pltpu.sync_copy(colb_hbm.at[pl.ds(blk, CHB)], col_v)
        pltpu.sync_copy(wb_hbm.at[pl.ds(blk, CHB)], w_v)
        for t in range(CHB):
            pltpu.sync_copy(w_v.at[t], acc_sh.at[col_v.at[t]], add=True)
        return carry
    lax.fori_loop(0, nchunks, chunk, 0)
    plsc.subcore_barrier()

    pltpu.sync_copy(acc_sh.at[pl.ds(s * NPT, NPT)], stage_v)
    pltpu.sync_copy(stage_v, out_hbm.at[c, pl.ds(s * NPT, NPT)])


@functools.partial(
    pl.kernel,
    out_type=jax.ShapeDtypeStruct((NQ, N, QW), jnp.float32),
    mesh=_mesh,
    scratch_types=[
        pltpu.VMEM((2, CHB, LPB), jnp.int32),
        pltpu.VMEM((2, CHB, LPB), jnp.int32),
        pltpu.VMEM((2, K), jnp.float32),
        pltpu.VMEM((2, K, QW), jnp.float32),
        pltpu.VMEM_SHARED((N, QW), jnp.float32),
        pltpu.SemaphoreType.DMA,
    ],
    compiler_params=pltpu.CompilerParams(use_tc_tiling_on_sc=False),
)
def _agg_sc(rowb_hbm, colb_hbm, w_hbm, tab_hbm, out_hbm,
            row_v, col_v, w_v, rows_v, acc_sh, gsem):
    c = lax.axis_index("c")
    s = lax.axis_index("s")
    b0 = s * NPT
    base_blk = s * ((E // LPB) // NS)
    z = jnp.zeros((16,), jnp.float32)

    def prefetch(j, slot, v):
        blk = base_blk + j * CHB
        pltpu.sync_copy(rowb_hbm.at[v, pl.ds(blk, CHB)], row_v.at[slot])
        pltpu.sync_copy(colb_hbm.at[pl.ds(blk, CHB)], col_v.at[slot])
        pltpu.sync_copy(w_hbm.at[pl.ds(blk * LPB, K)], w_v.at[slot])

    def fire(slot):
        for t in range(CHB):
            pltpu.async_copy(tab_hbm.at[row_v.at[slot, t]],
                             rows_v.at[slot, pl.ds(t * LPB, LPB)], gsem)

    def wait_gathers(slot):
        for t in range(CHB):
            pltpu.make_async_copy(tab_hbm.at[row_v.at[slot, t]],
                                  rows_v.at[slot, pl.ds(t * LPB, LPB)],
                                  gsem).wait()

    def mul(slot):
        def body(g, carry):
            base = g * 16
            w16 = w_v[slot, pl.ds(base, 16)]
            for i in range(16):
                ws = jnp.broadcast_to(w16[i], (16,))
                e = base + i
                rows_v[slot, e, pl.ds(0, 16)] = \
                    rows_v[slot, e, pl.ds(0, 16)] * ws
            return carry
        lax.fori_loop(0, K // 16, body, 0)

    def scatter(slot):
        for t in range(CHB):
            pltpu.sync_copy(rows_v.at[slot, pl.ds(t * LPB, LPB)],
                            acc_sh.at[col_v.at[slot, t]], add=True)

    for q in range(2):
        v = c * 2 + q

        def zrow(i, carry):
            rows_v[0, i, pl.ds(0, 16)] = z
            return carry
        lax.fori_loop(0, K, zrow, 0)
        for p in range(3):
            pltpu.sync_copy(rows_v.at[0], acc_sh.at[pl.ds(b0 + p * K, K)])
        pltpu.sync_copy(rows_v.at[0, pl.ds(0, 128)],
                        acc_sh.at[pl.ds(b0 + 3 * K, 128)])
        plsc.subcore_barrier()

        prefetch(0, 0, v)
        fire(0)
        prefetch(1, 1, v)

        def pair(hh, carry):
            j0 = 2 * hh
            wait_gathers(0)
            mul(0)
            fire(1)
            scatter(0)
            prefetch(j0 + 2, 0, v)
            wait_gathers(1)
            mul(1)
            fire(0)
            scatter(1)
            prefetch(j0 + 3, 1, v)
            return carry
        lax.fori_loop(0, NCHUNK // 2 - 1, pair, 0)

        wait_gathers(0)
        mul(0)
        fire(1)
        scatter(0)
        wait_gathers(1)
        mul(1)
        scatter(1)
        plsc.subcore_barrier()

        for p in range(3):
            pltpu.sync_copy(acc_sh.at[pl.ds(b0 + p * K, K)], rows_v.at[0])
            pltpu.sync_copy(rows_v.at[0], out_hbm.at[v, pl.ds(b0 + p * K, K)])
        pltpu.sync_copy(acc_sh.at[pl.ds(b0 + 3 * K, 128)],
                        rows_v.at[0, pl.ds(0, 128)])
        pltpu.sync_copy(rows_v.at[0, pl.ds(0, 128)],
                        out_hbm.at[v, pl.ds(b0 + 3 * K, 128)])


_BN = 3200
_PH = lax.Precision.HIGHEST


def _mish(v):
    return v * jnp.tanh(jnp.logaddexp(v, 0.0))


def _dot(a, bT):
    return lax.dot_general(a, bT, (((1,), (1,)), ((), ())),
                           preferred_element_type=jnp.float32)


def _mm_body(x_ref, w_ref, o_ref):
    o_ref[...] = _dot(x_ref[...], w_ref[...])


def _mm(x, w):
    n, d = x.shape
    fo = w.shape[0]
    return pl.pallas_call(
        _mm_body,
        grid=(n // _BN,),
        in_specs=[pl.BlockSpec((_BN, d), lambda i: (i, 0)),
                  pl.BlockSpec((fo, d), lambda i: (0, 0))],
        out_specs=pl.BlockSpec((_BN, fo), lambda i: (i, 0)),
        out_shape=jax.ShapeDtypeStruct((n, fo), jnp.float32),
    )(x, w)


def _scale_body(p_ref, h_ref, o_ref):
    dinv = lax.rsqrt(p_ref[0] + p_ref[1] + 1.0)
    hs = h_ref[...] * dinv[:, None]
    for v in range(NQ):
        o_ref[v] = hs[:, v * QW:(v + 1) * QW]


def _scale(degp, h):
    return pl.pallas_call(
        _scale_body,
        grid=(N // _BN,),
        in_specs=[pl.BlockSpec((NC, _BN), lambda i: (0, i)),
                  pl.BlockSpec((_BN, H), lambda i: (i, 0))],
        out_specs=pl.BlockSpec((NQ, _BN, QW), lambda i: (0, i, 0)),
        out_shape=jax.ShapeDtypeStruct((NQ, N, QW), jnp.float32),
    )(degp, h)


def _layer_mid_body(p_ref, g_ref, h_ref, b_ref, w2_ref, oh_ref, os_ref):
    dinv = lax.rsqrt(p_ref[0] + p_ref[1] + 1.0)
    a = jnp.concatenate([g_ref[v] for v in range(NQ)], axis=1)
    n1 = _mish(a * dinv[:, None] + h_ref[...] * (dinv * dinv)[:, None]
               + b_ref[...])
    h2 = _dot(n1, w2_ref[...])
    oh_ref[...] = h2
    hs2 = h2 * dinv[:, None]
    for v in range(NQ):
        os_ref[v] = hs2[:, v * QW:(v + 1) * QW]


def _layer_mid(degp, agg, h, b, w2):
    return pl.pallas_call(
        _layer_mid_body,
        grid=(N // _BN,),
        in_specs=[pl.BlockSpec((NC, _BN), lambda i: (0, i)),
                  pl.BlockSpec((NQ, _BN, QW), lambda i: (0, i, 0)),
                  pl.BlockSpec((_BN, H), lambda i: (i, 0)),
                  pl.BlockSpec((1, H), lambda i: (0, 0)),
                  pl.BlockSpec((H, H), lambda i: (0, 0))],
        out_specs=[pl.BlockSpec((_BN, H), lambda i: (i, 0)),
                   pl.BlockSpec((NQ, _BN, QW), lambda i: (0, i, 0))],
        out_shape=[jax.ShapeDtypeStruct((N, H), jnp.float32),
                   jax.ShapeDtypeStruct((NQ, N, QW), jnp.float32)],
    )(degp, agg, h, b, w2)


def _layer_out_body(p_ref, g_ref, h_ref, b_ref, wro_ref, bro_ref, o_ref):
    dinv = lax.rsqrt(p_ref[0] + p_ref[1] + 1.0)
    a = jnp.concatenate([g_ref[v] for v in range(NQ)], axis=1)
    n2 = _mish(a * dinv[:, None] + h_ref[...] * (dinv * dinv)[:, None]
               + b_ref[...])
    o_ref[...] = _mish(_dot(n2, wro_ref[...]) + bro_ref[...])


def _layer_out(degp, agg, h, b, wro, bro):
    return pl.pallas_call(
        _layer_out_body,
        grid=(N // _BN,),
        in_specs=[pl.BlockSpec((NC, _BN), lambda i: (0, i)),
                  pl.BlockSpec((NQ, _BN, QW), lambda i: (0, i, 0)),
                  pl.BlockSpec((_BN, H), lambda i: (i, 0)),
                  pl.BlockSpec((1, H), lambda i: (0, 0)),
                  pl.BlockSpec((8, H), lambda i: (0, 0)),
                  pl.BlockSpec((1, 8), lambda i: (0, 0))],
        out_specs=pl.BlockSpec((_BN, 8), lambda i: (i, 0)),
        out_shape=jax.ShapeDtypeStruct((N, 8), jnp.float32),
    )(degp, agg, h, b, wro, bro)


def _head_body(f_ref, w1_ref, b1_ref, g_ref, be_ref, w2_ref, b2_ref,
               wd1_ref, bd1_ref, wd2_ref, bd2_ref, ol_ref, oc_ref):
    z = _dot(f_ref[...], w1_ref[...]) + b1_ref[...]
    mu = jnp.mean(z, axis=0, keepdims=True)
    var = jnp.mean((z - mu) ** 2, axis=0, keepdims=True)
    zn = (z - mu) / jnp.sqrt(var + 1e-5)
    mid = _mish(zn * g_ref[...] + be_ref[...])
    ol_ref[...] = _dot(mid, w2_ref[...]) + b2_ref[...]
    dd = jnp.maximum(_dot(mid, wd1_ref[...]) + bd1_ref[...], 0.0)
    oc_ref[...] = _dot(dd, wd2_ref[...]) + bd2_ref[...]


def _head(feats, wfc1, bfc1, gamma, beta, wfc2, bfc2, wd1, bd1, wd2, bd2):
    full = lambda s: pl.BlockSpec(s, lambda: tuple(0 for _ in s))
    return pl.pallas_call(
        _head_body,
        in_specs=[full((B, 8 * D)), full((D, 8 * D)), full((1, D)),
                  full((1, D)), full((1, D)), full((2, D)), full((1, 2)),
                  full((H, D)), full((1, H)), full((6, H)), full((1, 6))],
        out_specs=[full((B, 2)), full((B, 6))],
        out_shape=[jax.ShapeDtypeStruct((B, 2), jnp.float32),
                   jax.ShapeDtypeStruct((B, 6), jnp.float32)],
    )(feats, wfc1, bfc1, gamma, beta, wfc2, bfc2, wd1, bd1, wd2, bd2)


@jax.jit
def kernel(x, edge_index, edge_attr, batch, W1, b1, W2, b2, Wro, bro,
           Wfc1, bfc1, gamma, beta, Wfc2, bfc2, Wd1, bd1, Wd2, bd2):
    row = edge_index[0]
    col = edge_index[1]
    rowb = jnp.stack([row + v * N for v in range(NQ)]).reshape(
        NQ, E // LPB, LPB)
    colb = col.reshape(E // LPB, LPB)
    wb = edge_attr.reshape(E // LPB, LPB)

    degp = _deg_sc(colb, wb)
    h1 = _mm(x, W1)
    hs1 = _scale(degp, h1)
    agg1 = _agg_sc(rowb, colb, edge_attr, hs1.reshape(NQ * N, QW))
    h2, hs2 = _layer_mid(degp, agg1, h1, b1.reshape(1, H), W2)
    agg2 = _agg_sc(rowb, colb, edge_attr, hs2.reshape(NQ * N, QW))
    out8 = _layer_out(degp, agg2, h2, b2.reshape(1, H), Wro, bro.reshape(1, 8))
    feats = out8.reshape(B, 8 * D)
    logits, cls = _head(feats, Wfc1, bfc1.reshape(1, D), gamma.reshape(1, D),
                        beta.reshape(1, D), Wfc2, bfc2.reshape(1, 2),
                        Wd1, bd1.reshape(1, H), Wd2, bd2.reshape(1, 6))
    return (logits, cls)
